# baseline (speedup 1.0000x reference)
import functools

import jax
import jax.numpy as jnp
from jax import lax
from jax.experimental import pallas as pl
from jax.experimental.pallas import tpu as pltpu
from jax.experimental.pallas import tpu_sc as plsc

V = 1000
B = 1024
T = 50

_info = plsc.get_sparse_core_info()
NC = _info.num_cores
NS = _info.num_subcores
L = _info.num_lanes
NW = NC * NS
RPW = B // NW
PAIRS = RPW // 2
CP = 64


def _logp_body(table_ref, logp_ref):
    x = table_ref[...]
    m = jnp.max(x, axis=1, keepdims=True)
    s = jnp.sum(jnp.exp(x - m), axis=1, keepdims=True)
    logp_ref[...] = x - (m + jnp.log(s))


def _table_logp(table):
    return pl.pallas_call(
        _logp_body,
        out_shape=jax.ShapeDtypeStruct((V, V), jnp.float32),
    )(table)


_mesh = plsc.VectorSubcoreMesh(core_axis_name="c", subcore_axis_name="s")

_GROUPS = (0, 16, 32, 34)


@functools.partial(
    pl.kernel,
    out_type=(
        jax.ShapeDtypeStruct((B, T, V), jnp.float32),
        jax.ShapeDtypeStruct((NW, L), jnp.float32),
    ),
    mesh=_mesh,
    compiler_params=pltpu.CompilerParams(use_tc_tiling_on_sc=False),
    scratch_types=[
        pltpu.VMEM((2, CP), jnp.int32),
        pltpu.VMEM((2, CP), jnp.int32),
        pltpu.VMEM((2, CP), jnp.int32),
        pltpu.VMEM((2, T, V), jnp.float32),
        pltpu.VMEM((2, CP), jnp.float32),
        pltpu.VMEM((L,), jnp.float32),
        pltpu.SemaphoreType.DMA,
        pltpu.SemaphoreType.DMA,
        pltpu.SemaphoreType.DMA,
        pltpu.SemaphoreType.DMA,
    ],
)
def _sc_gather_nll(ixs_hbm, tgt_hbm, table_hbm, logp_hbm, out_hbm, part_hbm,
                   idx_v, tgt_v, lin_v, rows_v, val_v, acc_v,
                   sem_g0, sem_g1, sem_s0, sem_s1):
    wid = lax.axis_index("s") * NC + lax.axis_index("c")
    wrow0 = wid * RPW
    sems_g = (sem_g0, sem_g1)
    sems_s = (sem_s0, sem_s1)

    acc_v[...] = jnp.zeros((L,), jnp.float32)
    zero = jnp.zeros((L,), jnp.float32)
    for b in (0, 1):
        val_v.at[b][pl.ds(48, L)] = zero

    def start_gather(c, b):
        row = wrow0 + c
        pltpu.sync_copy(ixs_hbm.at[row], idx_v.at[b].at[pl.ds(0, T)])
        pltpu.sync_copy(tgt_hbm.at[row], tgt_v.at[b].at[pl.ds(0, T)])
        for off in _GROUPS:
            il = idx_v.at[b][pl.ds(off, L)]
            tl = tgt_v.at[b][pl.ds(off, L)]
            lin_v.at[b][pl.ds(off, L)] = il * V + tl
        pltpu.async_copy(table_hbm.at[idx_v.at[b].at[pl.ds(0, T)]],
                         rows_v.at[b], sems_g[b])
        pltpu.async_copy(logp_hbm.at[lin_v.at[b].at[pl.ds(0, T)]],
                         val_v.at[b].at[pl.ds(0, T)], sems_g[b])

    def wait_gather(b):
        pltpu.make_async_copy(table_hbm.at[idx_v.at[b].at[pl.ds(0, T)]],
                              rows_v.at[b], sems_g[b]).wait()
        pltpu.make_async_copy(logp_hbm.at[lin_v.at[b].at[pl.ds(0, T)]],
                              val_v.at[b].at[pl.ds(0, T)], sems_g[b]).wait()

    def wait_scatter(b):
        pltpu.make_async_copy(rows_v.at[b], out_hbm.at[0], sems_s[b]).wait()

    start_gather(0, 0)

    def pair(g, carry):
        for b in (0, 1):
            c = 2 * g + b
            nb = 1 - b
            wait_gather(b)
            pltpu.async_copy(rows_v.at[b], out_hbm.at[wrow0 + c], sems_s[b])
            for j in range(CP // L):
                acc_v[...] = acc_v[...] + val_v.at[b][pl.ds(j * L, L)]
            if b == 0:
                @pl.when(g > 0)
                def _():
                    wait_scatter(nb)
                start_gather(c + 1, nb)
            else:
                wait_scatter(nb)
                @pl.when(g < PAIRS - 1)
                def _():
                    start_gather(c + 1, nb)
        return carry

    lax.fori_loop(0, PAIRS, pair, 0)
    wait_scatter(1)
    pltpu.sync_copy(acc_v, part_hbm.at[wid])


def kernel(ixs, targets, table):
    b, t = ixs.shape
    logp = _table_logp(table).reshape(-1)
    logits, part = _sc_gather_nll(ixs, targets, table, logp)
    loss = -jnp.sum(part) / (b * t)
    return (logits, loss)


# speedup vs baseline: 1.1303x; 1.1303x over previous
import functools

import jax
import jax.numpy as jnp
from jax import lax
from jax.experimental import pallas as pl
from jax.experimental.pallas import tpu as pltpu
from jax.experimental.pallas import tpu_sc as plsc

V = 1000
B = 1024
T = 50
KT = 8

_info = plsc.get_sparse_core_info()
NC = _info.num_cores
NS = _info.num_subcores
L = _info.num_lanes
NW = NC * NS
RPW = B // NW
PAIRS = RPW // 2
CP = 64


def _logp_body(table_ref, logp_ref):
    x = table_ref[...]
    m = jnp.max(x, axis=1, keepdims=True)
    s = jnp.sum(jnp.exp(x - m), axis=1, keepdims=True)
    logp_ref[...] = x - (m + jnp.log(s))


def _table_logp(table):
    return pl.pallas_call(
        _logp_body,
        out_shape=jax.ShapeDtypeStruct((V, V), jnp.float32),
    )(table)


_BB = 8


def _untile_body(x_ref, o_ref):
    for k in range(KT - 1):
        o_ref[:, :, k * 128:(k + 1) * 128] = x_ref[:, :, k, :]
    o_ref[:, :, (KT - 1) * 128:V] = x_ref[:, :, KT - 1, :V - (KT - 1) * 128]


def _untile(x):
    return pl.pallas_call(
        _untile_body,
        grid=(B // _BB,),
        in_specs=[pl.BlockSpec((_BB, T, KT, 128), lambda i: (i, 0, 0, 0))],
        out_specs=pl.BlockSpec((_BB, T, V), lambda i: (i, 0, 0)),
        out_shape=jax.ShapeDtypeStruct((B, T, V), jnp.float32),
    )(x)


_mesh = plsc.VectorSubcoreMesh(core_axis_name="c", subcore_axis_name="s")

_GROUPS = (0, 16, 32, 34)


@functools.partial(
    pl.kernel,
    out_type=(
        jax.ShapeDtypeStruct((B, T, KT, 128), jnp.float32),
        jax.ShapeDtypeStruct((NW, L), jnp.float32),
    ),
    mesh=_mesh,
    compiler_params=pltpu.CompilerParams(use_tc_tiling_on_sc=False),
    scratch_types=[
        pltpu.VMEM((2, CP), jnp.int32),
        pltpu.VMEM((2, CP), jnp.int32),
        pltpu.VMEM((2, CP), jnp.int32),
        pltpu.VMEM((2, T, KT, 128), jnp.float32),
        pltpu.VMEM((2, CP), jnp.float32),
        pltpu.VMEM((L,), jnp.float32),
        pltpu.SemaphoreType.DMA,
        pltpu.SemaphoreType.DMA,
        pltpu.SemaphoreType.DMA,
        pltpu.SemaphoreType.DMA,
    ],
)
def _sc_gather_nll(ixs_hbm, tgt_hbm, table_hbm, logp_hbm, out_hbm, part_hbm,
                   idx_v, tgt_v, lin_v, rows_v, val_v, acc_v,
                   sem_g0, sem_g1, sem_s0, sem_s1):
    wid = lax.axis_index("s") * NC + lax.axis_index("c")
    wrow0 = wid * RPW
    sems_g = (sem_g0, sem_g1)
    sems_s = (sem_s0, sem_s1)

    acc_v[...] = jnp.zeros((L,), jnp.float32)
    zero = jnp.zeros((L,), jnp.float32)
    for b in (0, 1):
        val_v.at[b][pl.ds(48, L)] = zero

    def start_gather(c, b):
        row = wrow0 + c
        pltpu.sync_copy(ixs_hbm.at[row], idx_v.at[b].at[pl.ds(0, T)])
        pltpu.sync_copy(tgt_hbm.at[row], tgt_v.at[b].at[pl.ds(0, T)])
        for off in _GROUPS:
            il = idx_v.at[b][pl.ds(off, L)]
            tl = tgt_v.at[b][pl.ds(off, L)]
            lin_v.at[b][pl.ds(off, L)] = il * V + tl
        pltpu.async_copy(table_hbm.at[idx_v.at[b].at[pl.ds(0, T)]],
                         rows_v.at[b], sems_g[b])
        pltpu.async_copy(logp_hbm.at[lin_v.at[b].at[pl.ds(0, T)]],
                         val_v.at[b].at[pl.ds(0, T)], sems_g[b])

    def wait_gather(b):
        pltpu.make_async_copy(table_hbm.at[idx_v.at[b].at[pl.ds(0, T)]],
                              rows_v.at[b], sems_g[b]).wait()
        pltpu.make_async_copy(logp_hbm.at[lin_v.at[b].at[pl.ds(0, T)]],
                              val_v.at[b].at[pl.ds(0, T)], sems_g[b]).wait()

    def wait_scatter(b):
        pltpu.make_async_copy(rows_v.at[b], out_hbm.at[0], sems_s[b]).wait()

    start_gather(0, 0)

    def pair(g, carry):
        for b in (0, 1):
            c = 2 * g + b
            nb = 1 - b
            wait_gather(b)
            pltpu.async_copy(rows_v.at[b], out_hbm.at[wrow0 + c], sems_s[b])
            for j in range(CP // L):
                acc_v[...] = acc_v[...] + val_v.at[b][pl.ds(j * L, L)]
            if b == 0:
                @pl.when(g > 0)
                def _():
                    wait_scatter(nb)
                start_gather(c + 1, nb)
            else:
                wait_scatter(nb)
                @pl.when(g < PAIRS - 1)
                def _():
                    start_gather(c + 1, nb)
        return carry

    lax.fori_loop(0, PAIRS, pair, 0)
    wait_scatter(1)
    pltpu.sync_copy(acc_v, part_hbm.at[wid])


def kernel(ixs, targets, table):
    b, t = ixs.shape
    logp = _table_logp(table).reshape(-1)
    table3 = jnp.pad(table, ((0, 0), (0, KT * 128 - V))).reshape(V, KT, 128)
    rows4, part = _sc_gather_nll(ixs, targets, table3, logp)
    logits = _untile(rows4)
    loss = -jnp.sum(part) / (b * t)
    return (logits, loss)


# speedup vs baseline: 1.1929x; 1.0554x over previous
import functools

import jax
import jax.numpy as jnp
from jax import lax
from jax.experimental import pallas as pl
from jax.experimental.pallas import tpu as pltpu
from jax.experimental.pallas import tpu_sc as plsc

V = 1000
B = 1024
T = 50
KT = 8

_info = plsc.get_sparse_core_info()
NC = _info.num_cores
NS = _info.num_subcores
L = _info.num_lanes
NW = NC * NS
CP = 64

NCH = 4
BC = B // NCH
RPW = BC // NW
PAIRS = RPW // 2


def _logp_body(table_ref, logp_ref):
    x = table_ref[...]
    m = jnp.max(x, axis=1, keepdims=True)
    s = jnp.sum(jnp.exp(x - m), axis=1, keepdims=True)
    logp_ref[...] = x - (m + jnp.log(s))


def _table_logp(table):
    return pl.pallas_call(
        _logp_body,
        out_shape=jax.ShapeDtypeStruct((V, V), jnp.float32),
    )(table)


_BB = 8
_NBLK = BC // _BB


def _untile_first_body(x_ref, o_ref):
    for k in range(KT - 1):
        o_ref[:, :, k * 128:(k + 1) * 128] = x_ref[:, :, k, :]
    o_ref[:, :, (KT - 1) * 128:V] = x_ref[:, :, KT - 1, :V - (KT - 1) * 128]


def _untile_next_body(x_ref, _prev_ref, o_ref):
    _untile_first_body(x_ref, o_ref)


def _untile_chunk(prev, x, k):
    out_spec = pl.BlockSpec((_BB, T, V), lambda i, k=k: (k * _NBLK + i, 0, 0))
    x_spec = pl.BlockSpec((_BB, T, KT, 128), lambda i: (i, 0, 0, 0))
    out_shape = jax.ShapeDtypeStruct((B, T, V), jnp.float32)
    if prev is None:
        return pl.pallas_call(
            _untile_first_body,
            grid=(_NBLK,),
            in_specs=[x_spec],
            out_specs=out_spec,
            out_shape=out_shape,
        )(x)
    prev_spec = pl.BlockSpec((1, 8, 128), lambda i: (0, 0, 0))
    return pl.pallas_call(
        _untile_next_body,
        grid=(_NBLK,),
        in_specs=[x_spec, prev_spec],
        out_specs=out_spec,
        out_shape=out_shape,
        input_output_aliases={1: 0},
    )(x, prev)


_mesh = plsc.VectorSubcoreMesh(core_axis_name="c", subcore_axis_name="s")

_GROUPS = (0, 16, 32, 34)


@functools.partial(
    pl.kernel,
    out_type=(
        jax.ShapeDtypeStruct((BC, T, KT, 128), jnp.float32),
        jax.ShapeDtypeStruct((NW, L), jnp.float32),
    ),
    mesh=_mesh,
    compiler_params=pltpu.CompilerParams(use_tc_tiling_on_sc=False),
    scratch_types=[
        pltpu.VMEM((2, CP), jnp.int32),
        pltpu.VMEM((2, CP), jnp.int32),
        pltpu.VMEM((2, CP), jnp.int32),
        pltpu.VMEM((2, T, KT, 128), jnp.float32),
        pltpu.VMEM((2, CP), jnp.float32),
        pltpu.VMEM((L,), jnp.float32),
        pltpu.SemaphoreType.DMA,
        pltpu.SemaphoreType.DMA,
        pltpu.SemaphoreType.DMA,
        pltpu.SemaphoreType.DMA,
    ],
)
def _sc_gather_nll(ixs_hbm, tgt_hbm, table_hbm, logp_hbm, out_hbm, part_hbm,
                   idx_v, tgt_v, lin_v, rows_v, val_v, acc_v,
                   sem_g0, sem_g1, sem_s0, sem_s1):
    wid = lax.axis_index("s") * NC + lax.axis_index("c")
    wrow0 = wid * RPW
    sems_g = (sem_g0, sem_g1)
    sems_s = (sem_s0, sem_s1)

    acc_v[...] = jnp.zeros((L,), jnp.float32)
    zero = jnp.zeros((L,), jnp.float32)
    for b in (0, 1):
        val_v.at[b][pl.ds(48, L)] = zero

    def start_gather(c, b):
        row = wrow0 + c
        pltpu.sync_copy(ixs_hbm.at[row], idx_v.at[b].at[pl.ds(0, T)])
        pltpu.sync_copy(tgt_hbm.at[row], tgt_v.at[b].at[pl.ds(0, T)])
        for off in _GROUPS:
            il = idx_v.at[b][pl.ds(off, L)]
            tl = tgt_v.at[b][pl.ds(off, L)]
            lin_v.at[b][pl.ds(off, L)] = il * V + tl
        pltpu.async_copy(table_hbm.at[idx_v.at[b].at[pl.ds(0, T)]],
                         rows_v.at[b], sems_g[b])
        pltpu.async_copy(logp_hbm.at[lin_v.at[b].at[pl.ds(0, T)]],
                         val_v.at[b].at[pl.ds(0, T)], sems_g[b])

    def wait_gather(b):
        pltpu.make_async_copy(table_hbm.at[idx_v.at[b].at[pl.ds(0, T)]],
                              rows_v.at[b], sems_g[b]).wait()
        pltpu.make_async_copy(logp_hbm.at[lin_v.at[b].at[pl.ds(0, T)]],
                              val_v.at[b].at[pl.ds(0, T)], sems_g[b]).wait()

    def wait_scatter(b):
        pltpu.make_async_copy(rows_v.at[b], out_hbm.at[0], sems_s[b]).wait()

    start_gather(0, 0)

    def pair(g, carry):
        for b in (0, 1):
            c = 2 * g + b
            nb = 1 - b
            wait_gather(b)
            pltpu.async_copy(rows_v.at[b], out_hbm.at[wrow0 + c], sems_s[b])
            for j in range(CP // L):
                acc_v[...] = acc_v[...] + val_v.at[b][pl.ds(j * L, L)]
            if b == 0:
                @pl.when(g > 0)
                def _():
                    wait_scatter(nb)
                start_gather(c + 1, nb)
            else:
                wait_scatter(nb)
                @pl.when(g < PAIRS - 1)
                def _():
                    start_gather(c + 1, nb)
        return carry

    lax.fori_loop(0, PAIRS, pair, 0)
    wait_scatter(1)
    pltpu.sync_copy(acc_v, part_hbm.at[wid])


def kernel(ixs, targets, table):
    b, t = ixs.shape
    logp = _table_logp(table).reshape(-1)
    table3 = jnp.pad(table, ((0, 0), (0, KT * 128 - V))).reshape(V, KT, 128)
    chunks = [
        _sc_gather_nll(ixs[k * BC:(k + 1) * BC],
                       targets[k * BC:(k + 1) * BC], table3, logp)
        for k in range(NCH)
    ]
    logits = None
    for k, (rows4, _) in enumerate(chunks):
        logits = _untile_chunk(logits, rows4, k)
    loss = -sum(jnp.sum(p) for _, p in chunks) / (b * t)
    return (logits, loss)


# speedup vs baseline: 1.2240x; 1.0261x over previous
import functools

import jax
import jax.numpy as jnp
from jax import lax
from jax.experimental import pallas as pl
from jax.experimental.pallas import tpu as pltpu
from jax.experimental.pallas import tpu_sc as plsc

V = 1000
B = 1024
T = 50
KT = 8

_info = plsc.get_sparse_core_info()
NC = _info.num_cores
NS = _info.num_subcores
L = _info.num_lanes
NW = NC * NS
CP = 64

NCH = 4
BC = B // NCH
RPW = BC // NW
PAIRS = RPW // 2


def _logp_body(table_ref, logp_ref):
    x = table_ref[...]
    m = jnp.max(x, axis=1, keepdims=True)
    s = jnp.sum(jnp.exp(x - m), axis=1, keepdims=True)
    logp_ref[...] = x - (m + jnp.log(s))


def _table_logp(table):
    return pl.pallas_call(
        _logp_body,
        out_shape=jax.ShapeDtypeStruct((V, V), jnp.float32),
    )(table)


_BB = 16
_NBLK = BC // _BB


def _untile_first_body(x_ref, o_ref):
    for k in range(KT - 1):
        o_ref[:, :, k * 128:(k + 1) * 128] = x_ref[:, :, k, :]
    o_ref[:, :, (KT - 1) * 128:V] = x_ref[:, :, KT - 1, :V - (KT - 1) * 128]


def _untile_next_body(x_ref, _prev_ref, o_ref):
    _untile_first_body(x_ref, o_ref)


def _untile_chunk(prev, x, k):
    out_spec = pl.BlockSpec((_BB, T, V), lambda i, k=k: (k * _NBLK + i, 0, 0))
    x_spec = pl.BlockSpec((_BB, T, KT, 128), lambda i: (i, 0, 0, 0))
    out_shape = jax.ShapeDtypeStruct((B, T, V), jnp.float32)
    if prev is None:
        return pl.pallas_call(
            _untile_first_body,
            grid=(_NBLK,),
            in_specs=[x_spec],
            out_specs=out_spec,
            out_shape=out_shape,
        )(x)
    prev_spec = pl.BlockSpec((1, 8, 128), lambda i: (0, 0, 0))
    return pl.pallas_call(
        _untile_next_body,
        grid=(_NBLK,),
        in_specs=[x_spec, prev_spec],
        out_specs=out_spec,
        out_shape=out_shape,
        input_output_aliases={1: 0},
    )(x, prev)


_mesh = plsc.VectorSubcoreMesh(core_axis_name="c", subcore_axis_name="s")

_GROUPS = (0, 16, 32, 34)


@functools.partial(
    pl.kernel,
    out_type=(
        jax.ShapeDtypeStruct((BC, T, KT, 128), jnp.float32),
        jax.ShapeDtypeStruct((NW, L), jnp.float32),
    ),
    mesh=_mesh,
    compiler_params=pltpu.CompilerParams(use_tc_tiling_on_sc=False),
    scratch_types=[
        pltpu.VMEM((2, CP), jnp.int32),
        pltpu.VMEM((2, CP), jnp.int32),
        pltpu.VMEM((2, CP), jnp.int32),
        pltpu.VMEM((2, T, KT, 128), jnp.float32),
        pltpu.VMEM((2, CP), jnp.float32),
        pltpu.VMEM((L,), jnp.float32),
        pltpu.SemaphoreType.DMA,
        pltpu.SemaphoreType.DMA,
        pltpu.SemaphoreType.DMA,
        pltpu.SemaphoreType.DMA,
    ],
)
def _sc_gather_nll(ixs_hbm, tgt_hbm, table_hbm, logp_hbm, out_hbm, part_hbm,
                   idx_v, tgt_v, lin_v, rows_v, val_v, acc_v,
                   sem_g0, sem_g1, sem_s0, sem_s1):
    wid = lax.axis_index("s") * NC + lax.axis_index("c")
    wrow0 = wid * RPW
    sems_g = (sem_g0, sem_g1)
    sems_s = (sem_s0, sem_s1)

    acc_v[...] = jnp.zeros((L,), jnp.float32)
    zero = jnp.zeros((L,), jnp.float32)
    for b in (0, 1):
        val_v.at[b][pl.ds(48, L)] = zero

    def start_gather(c, b):
        row = wrow0 + c
        pltpu.sync_copy(ixs_hbm.at[row], idx_v.at[b].at[pl.ds(0, T)])
        pltpu.sync_copy(tgt_hbm.at[row], tgt_v.at[b].at[pl.ds(0, T)])
        for off in _GROUPS:
            il = idx_v.at[b][pl.ds(off, L)]
            tl = tgt_v.at[b][pl.ds(off, L)]
            lin_v.at[b][pl.ds(off, L)] = il * V + tl
        pltpu.async_copy(table_hbm.at[idx_v.at[b].at[pl.ds(0, T)]],
                         rows_v.at[b], sems_g[b])
        pltpu.async_copy(logp_hbm.at[lin_v.at[b].at[pl.ds(0, T)]],
                         val_v.at[b].at[pl.ds(0, T)], sems_g[b])

    def wait_gather(b):
        pltpu.make_async_copy(table_hbm.at[idx_v.at[b].at[pl.ds(0, T)]],
                              rows_v.at[b], sems_g[b]).wait()
        pltpu.make_async_copy(logp_hbm.at[lin_v.at[b].at[pl.ds(0, T)]],
                              val_v.at[b].at[pl.ds(0, T)], sems_g[b]).wait()

    def wait_scatter(b):
        pltpu.make_async_copy(rows_v.at[b], out_hbm.at[0], sems_s[b]).wait()

    start_gather(0, 0)

    def pair(g, carry):
        for b in (0, 1):
            c = 2 * g + b
            nb = 1 - b
            wait_gather(b)
            pltpu.async_copy(rows_v.at[b], out_hbm.at[wrow0 + c], sems_s[b])
            for j in range(CP // L):
                acc_v[...] = acc_v[...] + val_v.at[b][pl.ds(j * L, L)]
            if b == 0:
                @pl.when(g > 0)
                def _():
                    wait_scatter(nb)
                start_gather(c + 1, nb)
            else:
                wait_scatter(nb)
                @pl.when(g < PAIRS - 1)
                def _():
                    start_gather(c + 1, nb)
        return carry

    lax.fori_loop(0, PAIRS, pair, 0)
    wait_scatter(1)
    pltpu.sync_copy(acc_v, part_hbm.at[wid])


def kernel(ixs, targets, table):
    b, t = ixs.shape
    logp = _table_logp(table).reshape(-1)
    table3 = jnp.pad(table, ((0, 0), (0, KT * 128 - V))).reshape(V, KT, 128)
    chunks = [
        _sc_gather_nll(ixs[k * BC:(k + 1) * BC],
                       targets[k * BC:(k + 1) * BC], table3, logp)
        for k in range(NCH)
    ]
    logits = None
    for k, (rows4, _) in enumerate(chunks):
        logits = _untile_chunk(logits, rows4, k)
    loss = -sum(jnp.sum(p) for _, p in chunks) / (b * t)
    return (logits, loss)
